# Initial kernel scaffold; baseline (speedup 1.0000x reference)
import jax
import jax.numpy as jnp
from jax.experimental import pallas as pl


def kernel(x, w_t, b):
    raise NotImplementedError("write your pallas kernel here")


# speedup vs baseline: 1.1374x; 1.1374x over previous
import jax
import jax.numpy as jnp
from jax.experimental import pallas as pl
from jax.experimental.pallas import tpu as pltpu

_NUM_CLASS = 1000
_TILE_M = 1024


def _cdiv(a: int, b: int) -> int:
    return (a + b - 1) // b


def _linear_kernel(x_ref, w_ref, b_ref, o_ref):
    xb = x_ref[...].astype(jnp.bfloat16)
    wb = w_ref[...].astype(jnp.bfloat16)
    acc = jnp.dot(xb, wb, preferred_element_type=jnp.float32)
    out = acc + b_ref[...]
    o_ref[...] = out[:, :_NUM_CLASS].astype(o_ref.dtype)


def kernel(x, w_t, b):
    B, D = x.shape
    Dw, Cp = w_t.shape
    assert D == Dw and _NUM_CLASS <= Cp

    tile_m = min(_TILE_M, B)
    grid = (_cdiv(B, tile_m),)
    return pl.pallas_call(
        _linear_kernel,
        out_shape=jax.ShapeDtypeStruct((B, _NUM_CLASS), x.dtype),
        grid=grid,
        in_specs=[
            pl.BlockSpec((tile_m, D), lambda i: (i, 0)),
            pl.BlockSpec((D, Cp), lambda i: (0, 0)),
            pl.BlockSpec((1, Cp), lambda i: (0, 0)),
        ],
        out_specs=pl.BlockSpec((tile_m, _NUM_CLASS), lambda i: (i, 0)),
        compiler_params=pltpu.CompilerParams(
            dimension_semantics=("parallel",)),
    )(x, w_t, b)
